# baseline (speedup 1.0000x reference)
import jax
import jax.numpy as jnp
import numpy as np
from jax.experimental import pallas as pl
from jax.experimental.pallas import tpu as pltpu

NUM_EXPERTS = 16
HIDDEN = 768
CAP_FACTORS = [1.5, 1.5, 1.5, 1.5, 1.0, 1.0, 1.0, 1.0, 1.0, 1.0, 1.0, 1.0, 0.5, 0.5, 0.5, 0.5]
BASE_CAP = 128
MAX_CAP = int(max(CAP_FACTORS) * BASE_CAP)


def _fused_kernel(x_ref, w_ref, b_ref, caps_ref,
                  disp_ref, comb_ref, zsum_ref,
                  work_ref, valsT_ref, idxT_ref):
    i = pl.program_id(0)
    R, T = work_ref.shape
    C = MAX_CAP
    G = x_ref.shape[0]
    E = R // G

    @pl.when(i == 0)
    def _select():
        w = w_ref[...]
        zsum = jnp.zeros((1, 1), jnp.float32)
        for g in range(G):
            xg = x_ref[g]
            logits = jax.lax.dot_general(
                w, xg, (((1,), (1,)), ((), ())),
                preferred_element_type=jnp.float32)
            logits = logits + b_ref[...].T
            m = jnp.max(logits, axis=0, keepdims=True)
            e = jnp.exp(logits - m)
            s = jnp.sum(e, axis=0, keepdims=True)
            work_ref[g * E:(g + 1) * E, :] = e / s
            logz = m + jnp.log(s)
            zsum = zsum + jnp.sum(logz * logz).reshape(1, 1)
        zsum_ref[...] = zsum

        NCH = 16
        CHW = T // NCH
        iota_c = jax.lax.broadcasted_iota(jnp.int32, (R, C), 1)
        iota_k = jax.lax.broadcasted_iota(jnp.int32, (R, NCH), 1)
        iota_k3 = jax.lax.broadcasted_iota(jnp.int32, (R, NCH, CHW), 1)
        iota_l3 = jax.lax.broadcasted_iota(jnp.int32, (R, NCH, CHW), 2)
        iota_l2 = jax.lax.broadcasted_iota(jnp.int32, (R, CHW), 1)
        UNROLL = 8

        def body(c, carry):
            vals, idxs, cur3, M = carry
            for u in range(UNROLL):
                mx = jnp.max(M, axis=1, keepdims=True)
                k = jnp.argmax(M, axis=1)[:, None]
                mask16 = iota_k == k
                mask3 = iota_k3 == k[:, :, None]
                w = jnp.sum(jnp.where(mask3, cur3, 0.0), axis=1)
                l = jnp.argmax(w, axis=1)[:, None]
                idx = k * CHW + l
                sel = iota_c == UNROLL * c + u
                vals = jnp.where(sel, mx, vals)
                idxs = jnp.where(sel, idx, idxs)
                kill = jnp.logical_and(mask3, iota_l3 == l[:, :, None])
                cur3 = jnp.where(kill, -jnp.inf, cur3)
                w2 = jnp.where(iota_l2 == l, -jnp.inf, w)
                M = jnp.where(mask16, jnp.max(w2, axis=1, keepdims=True), M)
            return (vals, idxs, cur3, M)

        vals0 = jnp.zeros((R, C), jnp.float32)
        idx0 = jnp.zeros((R, C), jnp.int32)
        cur0 = work_ref[...].reshape(R, NCH, CHW)
        m0 = jnp.max(cur0, axis=2)
        vals, idxs, _, _ = jax.lax.fori_loop(
            0, C // UNROLL, body, (vals0, idx0, cur0, m0))

        caps = caps_ref[:, 0:1]
        live = iota_c < caps
        valsT_ref[...] = jnp.where(live, vals, 0.0).T
        idxT_ref[...] = jnp.where(live, idxs, -1).T

    @pl.when(i > 0)
    def _materialize():
        r = i - 1
        lane_r = jax.lax.broadcasted_iota(jnp.int32, (C, R), 1)
        pick = lane_r == r
        val_col = jnp.sum(jnp.where(pick, valsT_ref[...], 0.0),
                          axis=1, keepdims=True)
        idx_col = jnp.sum(jnp.where(pick, idxT_ref[...], 0),
                          axis=1, keepdims=True)
        tid = jax.lax.broadcasted_iota(jnp.int32, (1, T), 1)
        hit = idx_col == tid
        comb_ref[0, 0] = jnp.where(hit, val_col, 0.0)
        disp_ref[0, 0] = hit


def kernel(token_inputs, W, b, expert_capacity):
    x = token_inputs.astype(jnp.float32)
    G, T, H = x.shape
    E = NUM_EXPERTS
    C = MAX_CAP
    R = G * E

    factors = jnp.asarray(CAP_FACTORS, dtype=jnp.float32)
    caps = jnp.floor(factors * expert_capacity).astype(jnp.int32)
    caps_rows = jnp.broadcast_to(jnp.tile(caps, G)[:, None], (R, 128))

    def _ge(i):
        r = jnp.maximum(i - 1, 0)
        return (r // E, r % E, 0, 0)

    disp_ect, comb_ect, zsum = pl.pallas_call(
        _fused_kernel,
        grid=(1 + G * E,),
        in_specs=[
            pl.BlockSpec((G, T, H), lambda i: (0, 0, 0)),
            pl.BlockSpec((E, H), lambda i: (0, 0)),
            pl.BlockSpec((1, E), lambda i: (0, 0)),
            pl.BlockSpec((R, 128), lambda i: (0, 0)),
        ],
        out_specs=[
            pl.BlockSpec((1, 1, C, T), _ge),
            pl.BlockSpec((1, 1, C, T), _ge),
            pl.BlockSpec((1, 1), lambda i: (0, 0)),
        ],
        out_shape=[
            jax.ShapeDtypeStruct((G, E, C, T), jnp.bool_),
            jax.ShapeDtypeStruct((G, E, C, T), jnp.float32),
            jax.ShapeDtypeStruct((1, 1), jnp.float32),
        ],
        scratch_shapes=[
            pltpu.VMEM((R, T), jnp.float32),
            pltpu.VMEM((C, R), jnp.float32),
            pltpu.VMEM((C, R), jnp.int32),
        ],
    )(x, W, b.reshape(1, E), caps_rows)

    router_z_loss = (zsum[0, 0] / (G * T)).astype(jnp.float32)
    auxiliary_loss = jnp.zeros((), dtype=jnp.float32)

    dispatch_mask = jnp.transpose(disp_ect, (0, 3, 1, 2))
    combine_array = jnp.transpose(comb_ect, (0, 3, 1, 2))
    return (dispatch_mask, combine_array, auxiliary_loss, router_z_loss)


# speedup vs baseline: 1.0663x; 1.0663x over previous
import jax
import jax.numpy as jnp
import numpy as np
from jax.experimental import pallas as pl
from jax.experimental.pallas import tpu as pltpu

NUM_EXPERTS = 16
HIDDEN = 768
CAP_FACTORS = [1.5, 1.5, 1.5, 1.5, 1.0, 1.0, 1.0, 1.0, 1.0, 1.0, 1.0, 1.0, 0.5, 0.5, 0.5, 0.5]
BASE_CAP = 128
MAX_CAP = int(max(CAP_FACTORS) * BASE_CAP)


def _fused_kernel(x_ref, w_ref, b_ref, caps_ref,
                  disp_ref, comb_ref, zsum_ref,
                  work_ref, valsT_ref, idxT_ref):
    i = pl.program_id(0)
    R, T = work_ref.shape
    C = MAX_CAP
    G = x_ref.shape[0]
    E = R // G

    @pl.when(i == 0)
    def _select():
        w = w_ref[...]
        zsum = jnp.zeros((1, 1), jnp.float32)
        for g in range(G):
            xg = x_ref[g]
            logits = jax.lax.dot_general(
                w, xg, (((1,), (1,)), ((), ())),
                preferred_element_type=jnp.float32)
            logits = logits + b_ref[...].T
            m = jnp.max(logits, axis=0, keepdims=True)
            e = jnp.exp(logits - m)
            s = jnp.sum(e, axis=0, keepdims=True)
            work_ref[g * E:(g + 1) * E, :] = e / s
            logz = m + jnp.log(s)
            zsum = zsum + jnp.sum(logz * logz).reshape(1, 1)
        zsum_ref[...] = zsum

        iota_t = jax.lax.broadcasted_iota(jnp.int32, (R, T), 1)
        iota_c = jax.lax.broadcasted_iota(jnp.int32, (R, C), 1)
        UNROLL = 8

        def body(c, carry):
            vals, idxs, cur = carry
            for u in range(UNROLL):
                mx = jnp.max(cur, axis=1, keepdims=True)
                idx = jnp.argmax(cur, axis=1)[:, None]
                sel = iota_c == UNROLL * c + u
                vals = jnp.where(sel, mx, vals)
                idxs = jnp.where(sel, idx, idxs)
                cur = jnp.where(iota_t == idx, -jnp.inf, cur)
            return (vals, idxs, cur)

        vals0 = jnp.zeros((R, C), jnp.float32)
        idx0 = jnp.zeros((R, C), jnp.int32)
        vals, idxs, _ = jax.lax.fori_loop(
            0, C // UNROLL, body, (vals0, idx0, work_ref[...]))

        caps = caps_ref[:, 0:1]
        live = iota_c < caps
        valsT_ref[...] = jnp.where(live, vals, 0.0).T
        idxT_ref[...] = jnp.where(live, idxs, -1).T

    @pl.when(i > 0)
    def _materialize():
        r = i - 1
        lane_r = jax.lax.broadcasted_iota(jnp.int32, (C, R), 1)
        pick = lane_r == r
        val_col = jnp.sum(jnp.where(pick, valsT_ref[...], 0.0),
                          axis=1, keepdims=True)
        idx_col = jnp.sum(jnp.where(pick, idxT_ref[...], 0),
                          axis=1, keepdims=True)
        tid = jax.lax.broadcasted_iota(jnp.int32, (1, T), 1)
        hit = idx_col == tid
        comb_ref[0, 0] = jnp.where(hit, val_col, 0.0)
        disp_ref[0, 0] = hit


def kernel(token_inputs, W, b, expert_capacity):
    x = token_inputs.astype(jnp.float32)
    G, T, H = x.shape
    E = NUM_EXPERTS
    C = MAX_CAP
    R = G * E

    factors = jnp.asarray(CAP_FACTORS, dtype=jnp.float32)
    caps = jnp.floor(factors * expert_capacity).astype(jnp.int32)
    caps_rows = jnp.broadcast_to(jnp.tile(caps, G)[:, None], (R, 128))

    def _ge(i):
        r = jnp.maximum(i - 1, 0)
        return (r // E, r % E, 0, 0)

    disp_ect, comb_ect, zsum = pl.pallas_call(
        _fused_kernel,
        grid=(1 + G * E,),
        in_specs=[
            pl.BlockSpec((G, T, H), lambda i: (0, 0, 0)),
            pl.BlockSpec((E, H), lambda i: (0, 0)),
            pl.BlockSpec((1, E), lambda i: (0, 0)),
            pl.BlockSpec((R, 128), lambda i: (0, 0)),
        ],
        out_specs=[
            pl.BlockSpec((1, 1, C, T), _ge),
            pl.BlockSpec((1, 1, C, T), _ge),
            pl.BlockSpec((1, 1), lambda i: (0, 0)),
        ],
        out_shape=[
            jax.ShapeDtypeStruct((G, E, C, T), jnp.bool_),
            jax.ShapeDtypeStruct((G, E, C, T), jnp.float32),
            jax.ShapeDtypeStruct((1, 1), jnp.float32),
        ],
        scratch_shapes=[
            pltpu.VMEM((R, T), jnp.float32),
            pltpu.VMEM((C, R), jnp.float32),
            pltpu.VMEM((C, R), jnp.int32),
        ],
    )(x, W, b.reshape(1, E), caps_rows)

    router_z_loss = (zsum[0, 0] / (G * T)).astype(jnp.float32)
    auxiliary_loss = jnp.zeros((), dtype=jnp.float32)

    dispatch_mask = jnp.transpose(disp_ect, (0, 3, 1, 2))
    combine_array = jnp.transpose(comb_ect, (0, 3, 1, 2))
    return (dispatch_mask, combine_array, auxiliary_loss, router_z_loss)
